# Initial kernel scaffold; baseline (speedup 1.0000x reference)
import jax
import jax.numpy as jnp
from jax.experimental import pallas as pl


def kernel(x, relative_pe, seq_len):
    raise NotImplementedError("write your pallas kernel here")


# speedup vs baseline: 2.8401x; 2.8401x over previous
import functools

import jax
import jax.numpy as jnp
from jax import lax
from jax.experimental import pallas as pl
from jax.experimental.pallas import tpu as pltpu
from jax.experimental.pallas import tpu_sc as plsc

NC = 2
NS = 16
L = 16


def _bias_kernel(S, D, n_rows):
    max_len = (n_rows + 1) // 2
    offset = max_len - 1
    NW = NC * NS
    R = S // NW
    J = 256
    n_chunks = S // J
    win = J + R - 1
    win_pad = ((win + L - 1) // L) * L

    mesh = plsc.VectorSubcoreMesh(
        core_axis_name="c", subcore_axis_name="s",
        num_cores=NC, num_subcores=NS,
    )

    @functools.partial(
        pl.kernel,
        out_type=jax.ShapeDtypeStruct((S, S, D), jnp.float32),
        mesh=mesh,
        compiler_params=pltpu.CompilerParams(use_tc_tiling_on_sc=False),
        scratch_types=[
            pltpu.VMEM((win_pad,), jnp.int32),
            pltpu.VMEM((win_pad, D), jnp.float32),
            pltpu.SemaphoreType.DMA,
        ],
    )
    def bias(pe_hbm, out_hbm, idx_v, win_v, sem):
        wid = lax.axis_index("s") * NC + lax.axis_index("c")
        i0 = wid * R
        for c in range(n_chunks):
            j0 = c * J
            base = i0 + (R - 1) - j0 + offset
            for t in range(win_pad // L):
                idx_v[pl.ds(t * L, L)] = (
                    jnp.full((L,), base - t * L, jnp.int32)
                    - lax.iota(jnp.int32, L)
                )
            copies = []
            k = 0
            while k < win_pad:
                n = min(128, win_pad - k)
                copies.append(pltpu.async_copy(
                    pe_hbm.at[idx_v.at[pl.ds(k, n)]],
                    win_v.at[pl.ds(k, n), :],
                    sem,
                ))
                k += n
            for cp in copies:
                cp.wait()
            for r in range(R):
                pltpu.sync_copy(
                    win_v.at[pl.ds(R - 1 - r, J), :],
                    out_hbm.at[i0 + r, pl.ds(j0, J), :],
                )

    return bias


def kernel(x, relative_pe, seq_len):
    S = x.shape[1]
    n_rows, D = relative_pe.shape
    return _bias_kernel(S, D, n_rows)(relative_pe)


# speedup vs baseline: 2.8462x; 1.0022x over previous
import functools

import jax
import jax.numpy as jnp
from jax import lax
from jax.experimental import pallas as pl
from jax.experimental.pallas import tpu as pltpu
from jax.experimental.pallas import tpu_sc as plsc

NC = 2
NS = 16
L = 16


def _bias_kernel(S, D, n_rows):
    max_len = (n_rows + 1) // 2
    offset = max_len - 1
    NW = NC * NS
    R = S // NW
    J = 256
    n_chunks = S // J
    win = J + R - 1
    win_pad = ((win + L - 1) // L) * L

    mesh = plsc.VectorSubcoreMesh(
        core_axis_name="c", subcore_axis_name="s",
        num_cores=NC, num_subcores=NS,
    )

    @functools.partial(
        pl.kernel,
        out_type=jax.ShapeDtypeStruct((S, S, D), jnp.float32),
        mesh=mesh,
        compiler_params=pltpu.CompilerParams(use_tc_tiling_on_sc=False),
        scratch_types=[
            pltpu.VMEM((win_pad,), jnp.int32),
            pltpu.VMEM((win_pad, D), jnp.float32),
            pltpu.SemaphoreType.DMA,
        ],
    )
    def bias(pe_hbm, out_hbm, idx_v, win_v, sem):
        wid = lax.axis_index("s") * NC + lax.axis_index("c")
        i0 = wid * R
        for c in range(n_chunks):
            j0 = c * J
            base = i0 + (R - 1) - j0 + offset
            for t in range(win_pad // L):
                idx_v[pl.ds(t * L, L)] = (
                    jnp.full((L,), base - t * L, jnp.int32)
                    - lax.iota(jnp.int32, L)
                )
            copies = []
            k = 0
            while k < win_pad:
                n = min(128, win_pad - k)
                copies.append(pltpu.async_copy(
                    pe_hbm.at[idx_v.at[pl.ds(k, n)]],
                    win_v.at[pl.ds(k, n), :],
                    sem,
                ))
                k += n
            for cp in copies:
                cp.wait()
            writes = []
            for r in range(R):
                writes.append(pltpu.async_copy(
                    win_v.at[pl.ds(R - 1 - r, J), :],
                    out_hbm.at[i0 + r, pl.ds(j0, J), :],
                    sem,
                ))
            for wr in writes:
                wr.wait()

    return bias


def kernel(x, relative_pe, seq_len):
    S = x.shape[1]
    n_rows, D = relative_pe.shape
    return _bias_kernel(S, D, n_rows)(relative_pe)


# speedup vs baseline: 4.3367x; 1.5237x over previous
import functools

import jax
import jax.numpy as jnp
from jax import lax
from jax.experimental import pallas as pl
from jax.experimental.pallas import tpu as pltpu
from jax.experimental.pallas import tpu_sc as plsc

NC = 2
NS = 16
L = 16
NSHIFT = 8


def _sc_gather_rev8(S, D, n_rows):
    max_len = (n_rows + 1) // 2
    offset = max_len - 1
    top = offset + S - 1
    NW = NC * NS
    W2 = 2 * S
    RW = NSHIFT * W2 // NW
    assert RW % L == 0 and W2 % RW == 0

    mesh = plsc.VectorSubcoreMesh(
        core_axis_name="c", subcore_axis_name="s",
        num_cores=NC, num_subcores=NS,
    )

    @functools.partial(
        pl.kernel,
        out_type=jax.ShapeDtypeStruct((NSHIFT * W2, D), jnp.float32),
        mesh=mesh,
        compiler_params=pltpu.CompilerParams(use_tc_tiling_on_sc=False),
        scratch_types=[
            pltpu.VMEM((RW,), jnp.int32),
            pltpu.VMEM((RW, D), jnp.float32),
            pltpu.SemaphoreType.DMA,
        ],
    )
    def gather(pe_hbm, rev8_hbm, idx_v, win_v, sem):
        wid = lax.axis_index("s") * NC + lax.axis_index("c")
        shift = wid // (W2 // RW)
        base = top + (W2 - 1) * shift - wid * RW
        for t in range(RW // L):
            idx_v[pl.ds(t * L, L)] = (
                jnp.full((L,), base - t * L, jnp.int32)
                - lax.iota(jnp.int32, L)
            )
        copies = []
        k = 0
        while k < RW:
            n = min(128, RW - k)
            copies.append(pltpu.async_copy(
                pe_hbm.at[idx_v.at[pl.ds(k, n)]],
                win_v.at[pl.ds(k, n), :],
                sem,
            ))
            k += n
        for cp in copies:
            cp.wait()
        pltpu.sync_copy(win_v, rev8_hbm.at[pl.ds(wid * RW, RW), :])

    return gather


def _tc_broadcast(S, D):
    W2 = 2 * S

    def body(rev_ref, out_ref):
        i = pl.program_id(0)
        t = (S - 1) - i
        s = lax.rem(t, NSHIFT)
        row = (W2 - 1) * s + t
        row = pl.multiple_of(row, NSHIFT)
        out_ref[0] = rev_ref[pl.ds(row, S), :]

    return pl.pallas_call(
        body,
        grid=(S,),
        in_specs=[pl.BlockSpec((NSHIFT * W2, D), lambda i: (0, 0))],
        out_specs=pl.BlockSpec((1, S, D), lambda i: (i, 0, 0)),
        out_shape=jax.ShapeDtypeStruct((S, S, D), jnp.float32),
        compiler_params=pltpu.CompilerParams(
            dimension_semantics=("arbitrary",),
            vmem_limit_bytes=64 * 1024 * 1024,
        ),
    )


def kernel(x, relative_pe, seq_len):
    S = x.shape[1]
    n_rows, D = relative_pe.shape
    rev8 = _sc_gather_rev8(S, D, n_rows)(relative_pe)
    return _tc_broadcast(S, D)(rev8)


# speedup vs baseline: 7.9916x; 1.8428x over previous
import functools

import jax
import jax.numpy as jnp
from jax import lax
from jax.experimental import pallas as pl
from jax.experimental.pallas import tpu as pltpu
from jax.experimental.pallas import tpu_sc as plsc

NC = 2
NS = 16
L = 16
NSHIFT = 8


def _sc_gather_rev8(S, D, n_rows):
    max_len = (n_rows + 1) // 2
    offset = max_len - 1
    top = offset + S - 1
    NW = NC * NS
    W2 = 2 * S
    RW = NSHIFT * W2 // NW
    assert RW % L == 0 and W2 % RW == 0

    mesh = plsc.VectorSubcoreMesh(
        core_axis_name="c", subcore_axis_name="s",
        num_cores=NC, num_subcores=NS,
    )

    @functools.partial(
        pl.kernel,
        out_type=jax.ShapeDtypeStruct((NSHIFT * W2, D), jnp.float32),
        mesh=mesh,
        compiler_params=pltpu.CompilerParams(use_tc_tiling_on_sc=False),
        scratch_types=[
            pltpu.VMEM((RW,), jnp.int32),
            pltpu.VMEM((RW, D), jnp.float32),
            pltpu.SemaphoreType.DMA,
        ],
    )
    def gather(pe_hbm, rev8_hbm, idx_v, win_v, sem):
        wid = lax.axis_index("s") * NC + lax.axis_index("c")
        shift = wid // (W2 // RW)
        base = top + (W2 - 1) * shift - wid * RW
        for t in range(RW // L):
            idx_v[pl.ds(t * L, L)] = (
                jnp.full((L,), base - t * L, jnp.int32)
                - lax.iota(jnp.int32, L)
            )
        copies = []
        k = 0
        while k < RW:
            n = min(128, RW - k)
            copies.append(pltpu.async_copy(
                pe_hbm.at[idx_v.at[pl.ds(k, n)]],
                win_v.at[pl.ds(k, n), :],
                sem,
            ))
            k += n
        for cp in copies:
            cp.wait()
        pltpu.sync_copy(win_v, rev8_hbm.at[pl.ds(wid * RW, RW), :])

    return gather


def _tc_broadcast(S, D, rows_per_step=4):
    W2 = 2 * S

    def body(rev_ref, out_ref):
        b = pl.program_id(0)
        for r in range(rows_per_step):
            i = b * rows_per_step + r
            t = (S - 1) - i
            s = lax.rem(t, NSHIFT)
            row = (W2 - 1) * s + t
            row = pl.multiple_of(row, NSHIFT)
            out_ref[r] = rev_ref[pl.ds(row, S), :]

    return pl.pallas_call(
        body,
        grid=(S // rows_per_step,),
        in_specs=[pl.BlockSpec((NSHIFT * W2, D), lambda i: (0, 0))],
        out_specs=pl.BlockSpec((rows_per_step, S, D), lambda i: (i, 0, 0)),
        out_shape=jax.ShapeDtypeStruct((S, S, D), jnp.float32),
        compiler_params=pltpu.CompilerParams(
            dimension_semantics=("arbitrary",),
            vmem_limit_bytes=64 * 1024 * 1024,
        ),
    )


def kernel(x, relative_pe, seq_len):
    S = x.shape[1]
    n_rows, D = relative_pe.shape
    rev8 = _sc_gather_rev8(S, D, n_rows)(relative_pe)
    return _tc_broadcast(S, D)(rev8)


# speedup vs baseline: 8.8911x; 1.1126x over previous
import functools

import jax
import jax.numpy as jnp
from jax import lax
from jax.experimental import pallas as pl
from jax.experimental.pallas import tpu as pltpu
from jax.experimental.pallas import tpu_sc as plsc

NC = 2
NS = 16
L = 16
NSHIFT = 8


def _sc_gather_rev8(S, D, n_rows):
    max_len = (n_rows + 1) // 2
    offset = max_len - 1
    top = offset + S - 1
    NW = NC * NS
    W2 = 2 * S
    RW = NSHIFT * W2 // NW
    assert RW % L == 0 and W2 % RW == 0

    mesh = plsc.VectorSubcoreMesh(
        core_axis_name="c", subcore_axis_name="s",
        num_cores=NC, num_subcores=NS,
    )

    @functools.partial(
        pl.kernel,
        out_type=jax.ShapeDtypeStruct((NSHIFT * W2, D), jnp.float32),
        mesh=mesh,
        compiler_params=pltpu.CompilerParams(use_tc_tiling_on_sc=False),
        scratch_types=[
            pltpu.VMEM((RW,), jnp.int32),
            pltpu.VMEM((RW, D), jnp.float32),
            pltpu.SemaphoreType.DMA,
        ],
    )
    def gather(pe_hbm, rev8_hbm, idx_v, win_v, sem):
        wid = lax.axis_index("s") * NC + lax.axis_index("c")
        shift = wid // (W2 // RW)
        base = top + (W2 - 1) * shift - wid * RW
        for t in range(RW // L):
            idx_v[pl.ds(t * L, L)] = (
                jnp.full((L,), base - t * L, jnp.int32)
                - lax.iota(jnp.int32, L)
            )
        copies = []
        k = 0
        while k < RW:
            n = min(128, RW - k)
            copies.append(pltpu.async_copy(
                pe_hbm.at[idx_v.at[pl.ds(k, n)]],
                win_v.at[pl.ds(k, n), :],
                sem,
            ))
            k += n
        for cp in copies:
            cp.wait()
        pltpu.sync_copy(win_v, rev8_hbm.at[pl.ds(wid * RW, RW), :])

    return gather


def _tc_broadcast(S, D, rows_per_step=8):
    W2 = 2 * S

    def body(rev_ref, out_ref):
        b = pl.program_id(0)
        for r in range(rows_per_step):
            i = b * rows_per_step + r
            t = (S - 1) - i
            s = lax.rem(t, NSHIFT)
            row = (W2 - 1) * s + t
            row = pl.multiple_of(row, NSHIFT)
            out_ref[r] = rev_ref[pl.ds(row, S), :]

    return pl.pallas_call(
        body,
        grid=(S // rows_per_step,),
        in_specs=[pl.BlockSpec((NSHIFT * W2, D), lambda i: (0, 0))],
        out_specs=pl.BlockSpec((rows_per_step, S, D), lambda i: (i, 0, 0)),
        out_shape=jax.ShapeDtypeStruct((S, S, D), jnp.float32),
        compiler_params=pltpu.CompilerParams(
            dimension_semantics=("arbitrary",),
            vmem_limit_bytes=64 * 1024 * 1024,
        ),
    )


def kernel(x, relative_pe, seq_len):
    S = x.shape[1]
    n_rows, D = relative_pe.shape
    rev8 = _sc_gather_rev8(S, D, n_rows)(relative_pe)
    return _tc_broadcast(S, D)(rev8)
